# baseline (speedup 1.0000x reference)
import functools

import jax
import jax.numpy as jnp
from jax import lax
from jax.experimental import pallas as pl
from jax.experimental.pallas import tpu as pltpu
from jax.experimental.pallas import tpu_sc as plsc

_PL, _PC = 8, 16
_FS = 100.0
_GR = 16
_NW = 32
_CHUNK = 128


def _sc_gather_body(L, C, xf_ref, sl_ref, sc_ref, out_ref,
                    slv, scv, eidx, patches, sem):
    num_cores = 2
    P = slv.shape[0]
    w = lax.axis_index("s") * num_cores + lax.axis_index("c")

    pltpu.sync_copy(sl_ref.at[pl.ds(w * P, P)], slv)
    pltpu.sync_copy(sc_ref.at[pl.ds(w * P, P)], scv)

    nchunks = P // 16
    base_w = w * (L * C)
    per_row = P // _CHUNK

    def build(c, _):
        sl16 = slv[pl.ds(c * 16, 16)]
        sc16 = scv[pl.ds(c * 16, 16)]
        base = base_w + sl16 * C + sc16
        col = (c % (_CHUNK // 16)) * 16
        rowoff = c // (_CHUNK // 16)
        for i in range(_PL):
            for j in range(_PC):
                k = i * _PC + j
                eidx[k * per_row + rowoff, pl.ds(col, 16)] = base + (i * C + j)
        return 0

    lax.fori_loop(0, nchunks, build, 0)

    ndma = (_PL * _PC * P) // _CHUNK
    group = 16
    handles = []
    for q in range(ndma):
        handles.append(pltpu.async_copy(
            xf_ref.at[eidx.at[q]],
            patches.at[pl.ds(q * _CHUNK, _CHUNK)], sem))
        if q >= group:
            handles[q - group].wait()
    for h in handles[ndma - group:]:
        h.wait()

    pltpu.sync_copy(patches, out_ref.at[w])


def _mlp_tc_kernel(sl2_ref, pt_ref, w1p_ref, w1t_ref, b1_ref,
                   w2t_ref, b2_ref, out_ref):
    P = out_ref.shape[1]
    slv = sl2_ref[0].astype(jnp.float32)
    iv = lax.broadcasted_iota(jnp.int32, (P, _PL), 1).astype(jnp.float32)
    tv = (slv + iv) * (1.0 / _FS)
    hi = lax.Precision.HIGHEST
    acc = jnp.dot(tv, w1t_ref[...], precision=hi) + b1_ref[...]
    acc += lax.dot_general(pt_ref[0], w1p_ref[...],
                           (((0,), (0,)), ((), ())), precision=hi)
    h = acc * jax.nn.sigmoid(acc)
    out_ref[0] = jnp.dot(h, w2t_ref[...], precision=hi) + b2_ref[...]


def kernel(x, start_indices_L, start_indices_C, W1, b1, W2, b2):
    B, L, C = x.shape
    P = start_indices_L.shape[1]
    D = W2.shape[0]
    BP = B * P

    sl = start_indices_L.astype(jnp.int32)
    sc = start_indices_C.astype(jnp.int32)
    xf = x.reshape(B * L * C)

    mesh = plsc.VectorSubcoreMesh(core_axis_name="c", subcore_axis_name="s",
                                  num_cores=2)
    sc_gather = functools.partial(
        pl.kernel, mesh=mesh,
        out_type=jax.ShapeDtypeStruct((B, _PL * _PC * P), jnp.float32),
        scratch_types=[
            pltpu.VMEM((P,), jnp.int32),
            pltpu.VMEM((P,), jnp.int32),
            pltpu.VMEM((_PL * _PC * P // _CHUNK, _CHUNK), jnp.int32),
            pltpu.VMEM((_PL * _PC * P,), jnp.float32),
            pltpu.SemaphoreType.DMA,
        ],
    )(functools.partial(_sc_gather_body, L, C))
    pt = sc_gather(xf, sl.reshape(BP), sc.reshape(BP))
    pt = pt.reshape(B, _PL * _PC, P)

    w1r = W1.reshape(D, _PL, 2 * _PC)
    w1p = w1r[:, :, :_PC].reshape(D, _PL * _PC).T
    w1t = w1r[:, :, _PC:].sum(axis=2).T
    w2t = W2.T
    b1r = b1.reshape(1, D)
    b2r = b2.reshape(1, D)

    out = pl.pallas_call(
        _mlp_tc_kernel,
        grid=(B,),
        in_specs=[
            pl.BlockSpec((1, P, 1), lambda b: (b, 0, 0)),
            pl.BlockSpec((1, _PL * _PC, P), lambda b: (b, 0, 0)),
            pl.BlockSpec((_PL * _PC, D), lambda b: (0, 0)),
            pl.BlockSpec((_PL, D), lambda b: (0, 0)),
            pl.BlockSpec((1, D), lambda b: (0, 0)),
            pl.BlockSpec((D, D), lambda b: (0, 0)),
            pl.BlockSpec((1, D), lambda b: (0, 0)),
        ],
        out_specs=pl.BlockSpec((1, P, D), lambda b: (b, 0, 0)),
        out_shape=jax.ShapeDtypeStruct((B, P, D), jnp.float32),
    )(sl.reshape(B, P, 1), pt, w1p, w1t, b1r, w2t, b2r)
    return out


# speedup vs baseline: 1.0289x; 1.0289x over previous
import functools

import jax
import jax.numpy as jnp
from jax import lax
from jax.experimental import pallas as pl
from jax.experimental.pallas import tpu as pltpu
from jax.experimental.pallas import tpu_sc as plsc

_PL, _PC = 8, 16
_FS = 100.0
_GR = 16
_NW = 32
_CHUNK = 128


def _sc_gather_body(L, C, xf_ref, sl_ref, sc_ref, out_ref,
                    slv, scv, eidx, patches, sem):
    num_cores = 2
    P = slv.shape[0]
    w = lax.axis_index("s") * num_cores + lax.axis_index("c")

    pltpu.sync_copy(sl_ref.at[pl.ds(w * P, P)], slv)
    pltpu.sync_copy(sc_ref.at[pl.ds(w * P, P)], scv)

    nchunks = P // 16
    base_w = w * (L * C)
    per_row = P // _CHUNK

    def build(c, _):
        sl16 = slv[pl.ds(c * 16, 16)]
        sc16 = scv[pl.ds(c * 16, 16)]
        base = base_w + sl16 * C + sc16
        col = (c % (_CHUNK // 16)) * 16
        rowoff = c // (_CHUNK // 16)
        for i in range(_PL):
            for j in range(_PC):
                k = i * _PC + j
                eidx[k * per_row + rowoff, pl.ds(col, 16)] = base + (i * C + j)
        return 0

    lax.fori_loop(0, nchunks, build, 0)

    ndma = (_PL * _PC * P) // _CHUNK
    group = 16
    handles = []
    for q in range(ndma):
        handles.append(pltpu.async_copy(
            xf_ref.at[eidx.at[q]],
            patches.at[pl.ds(q * _CHUNK, _CHUNK)], sem))
        if q >= group:
            handles[q - group].wait()
    for h in handles[ndma - group:]:
        h.wait()

    pltpu.sync_copy(patches, out_ref.at[w])


def _mlp_tc_kernel(sl2_ref, pt_ref, w1p_ref, w1t_ref, b1_ref,
                   w2t_ref, b2_ref, out_ref):
    B, P = out_ref.shape[0], out_ref.shape[1]
    iv = lax.broadcasted_iota(jnp.int32, (P, _PL), 1).astype(jnp.float32)
    hi = lax.Precision.HIGHEST
    w1t = w1t_ref[...]
    w1p = w1p_ref[...]
    w2t = w2t_ref[...]
    b1v = b1_ref[...]
    b2v = b2_ref[...]
    for b in range(B):
        slv = sl2_ref[b].astype(jnp.float32)
        tv = (slv + iv) * (1.0 / _FS)
        acc = jnp.dot(tv, w1t, precision=hi) + b1v
        acc += lax.dot_general(pt_ref[b], w1p,
                               (((0,), (0,)), ((), ())), precision=hi)
        h = acc * jax.nn.sigmoid(acc)
        out_ref[b] = jnp.dot(h, w2t, precision=hi) + b2v


def kernel(x, start_indices_L, start_indices_C, W1, b1, W2, b2):
    B, L, C = x.shape
    P = start_indices_L.shape[1]
    D = W2.shape[0]
    BP = B * P

    sl = start_indices_L.astype(jnp.int32)
    sc = start_indices_C.astype(jnp.int32)
    xf = x.reshape(B * L * C)

    mesh = plsc.VectorSubcoreMesh(core_axis_name="c", subcore_axis_name="s",
                                  num_cores=2)
    sc_gather = functools.partial(
        pl.kernel, mesh=mesh,
        out_type=jax.ShapeDtypeStruct((B, _PL * _PC * P), jnp.float32),
        scratch_types=[
            pltpu.VMEM((P,), jnp.int32),
            pltpu.VMEM((P,), jnp.int32),
            pltpu.VMEM((_PL * _PC * P // _CHUNK, _CHUNK), jnp.int32),
            pltpu.VMEM((_PL * _PC * P,), jnp.float32),
            pltpu.SemaphoreType.DMA,
        ],
    )(functools.partial(_sc_gather_body, L, C))
    pt = sc_gather(xf, sl.reshape(BP), sc.reshape(BP))
    pt = pt.reshape(B, _PL * _PC, P)

    w1r = W1.reshape(D, _PL, 2 * _PC)
    w1p = w1r[:, :, :_PC].reshape(D, _PL * _PC).T
    w1t = w1r[:, :, _PC:].sum(axis=2).T
    w2t = W2.T
    b1r = b1.reshape(1, D)
    b2r = b2.reshape(1, D)

    out = pl.pallas_call(
        _mlp_tc_kernel,
        out_shape=jax.ShapeDtypeStruct((B, P, D), jnp.float32),
    )(sl.reshape(B, P, 1), pt, w1p, w1t, b1r, w2t, b2r)
    return out


# speedup vs baseline: 1.1981x; 1.1644x over previous
import functools

import jax
import jax.numpy as jnp
from jax import lax
from jax.experimental import pallas as pl
from jax.experimental.pallas import tpu as pltpu
from jax.experimental.pallas import tpu_sc as plsc

_PL, _PC = 8, 16
_FS = 100.0
_GR = 16
_NW = 32
_CHUNK = 128


def _sc_gather_body(L, C, xf_ref, sl_ref, sc_ref, out_ref,
                    slv, scv, eidx, patches, sem):
    num_cores = 2
    P = slv.shape[0]
    w = lax.axis_index("s") * num_cores + lax.axis_index("c")

    pltpu.sync_copy(sl_ref.at[pl.ds(w * P, P)], slv)
    pltpu.sync_copy(sc_ref.at[pl.ds(w * P, P)], scv)

    nchunks = P // 16
    base_w = w * (L * C)
    per_row = P // _CHUNK

    def build(c, _):
        sl16 = slv[pl.ds(c * 16, 16)]
        sc16 = scv[pl.ds(c * 16, 16)]
        base = base_w + sl16 * C + sc16
        col = (c % (_CHUNK // 16)) * 16
        rowoff = c // (_CHUNK // 16)
        for i in range(_PL):
            for j in range(_PC):
                k = i * _PC + j
                eidx[k * per_row + rowoff, pl.ds(col, 16)] = base + (i * C + j)
        return 0

    lax.fori_loop(0, nchunks, build, 0)

    ndma = (_PL * _PC * P) // _CHUNK
    group = 16
    handles = []
    for q in range(ndma):
        handles.append(pltpu.async_copy(
            xf_ref.at[eidx.at[q]],
            patches.at[pl.ds(q * _CHUNK, _CHUNK)], sem))
        if q >= group:
            handles[q - group].wait()
    for h in handles[ndma - group:]:
        h.wait()

    pltpu.sync_copy(patches, out_ref.at[w])


def _mlp_tc_kernel(sl2_ref, pt_ref, w1p_ref, w1t_ref, b1_ref,
                   w2t_ref, b2_ref, out_ref):
    B, P = out_ref.shape[0], out_ref.shape[1]
    iv = lax.broadcasted_iota(jnp.int32, (P, _PL), 1).astype(jnp.float32)
    hi = lax.Precision.DEFAULT
    w1t = w1t_ref[...]
    w1p = w1p_ref[...]
    w2t = w2t_ref[...]
    b1v = b1_ref[...]
    b2v = b2_ref[...]
    for b in range(B):
        slv = sl2_ref[b].astype(jnp.float32)
        tv = (slv + iv) * (1.0 / _FS)
        acc = jnp.dot(tv, w1t, precision=hi) + b1v
        acc += lax.dot_general(pt_ref[b], w1p,
                               (((0,), (0,)), ((), ())), precision=hi)
        h = acc * jax.nn.sigmoid(acc)
        out_ref[b] = jnp.dot(h, w2t, precision=hi) + b2v


def kernel(x, start_indices_L, start_indices_C, W1, b1, W2, b2):
    B, L, C = x.shape
    P = start_indices_L.shape[1]
    D = W2.shape[0]
    BP = B * P

    sl = start_indices_L.astype(jnp.int32)
    sc = start_indices_C.astype(jnp.int32)
    xf = x.reshape(B * L * C)

    mesh = plsc.VectorSubcoreMesh(core_axis_name="c", subcore_axis_name="s",
                                  num_cores=2)
    sc_gather = functools.partial(
        pl.kernel, mesh=mesh,
        out_type=jax.ShapeDtypeStruct((B, _PL * _PC * P), jnp.float32),
        scratch_types=[
            pltpu.VMEM((P,), jnp.int32),
            pltpu.VMEM((P,), jnp.int32),
            pltpu.VMEM((_PL * _PC * P // _CHUNK, _CHUNK), jnp.int32),
            pltpu.VMEM((_PL * _PC * P,), jnp.float32),
            pltpu.SemaphoreType.DMA,
        ],
    )(functools.partial(_sc_gather_body, L, C))
    pt = sc_gather(xf, sl.reshape(BP), sc.reshape(BP))
    pt = pt.reshape(B, _PL * _PC, P)

    w1r = W1.reshape(D, _PL, 2 * _PC)
    w1p = w1r[:, :, :_PC].reshape(D, _PL * _PC).T
    w1t = w1r[:, :, _PC:].sum(axis=2).T
    w2t = W2.T
    b1r = b1.reshape(1, D)
    b2r = b2.reshape(1, D)

    out = pl.pallas_call(
        _mlp_tc_kernel,
        out_shape=jax.ShapeDtypeStruct((B, P, D), jnp.float32),
    )(sl.reshape(B, P, 1), pt, w1p, w1t, b1r, w2t, b2r)
    return out


# speedup vs baseline: 1.9759x; 1.6492x over previous
import functools

import jax
import jax.numpy as jnp
from jax import lax
from jax.experimental import pallas as pl
from jax.experimental.pallas import tpu as pltpu
from jax.experimental.pallas import tpu_sc as plsc

_PL, _PC = 8, 16
_FS = 100.0
_GR = 16
_NW = 32
_CHUNK = 128


def _sc_gather_body(L, C, xg_ref, sl_ref, sc_ref, out_ref,
                    slv, scv, gidx, rows, patches, sem):
    num_cores = 2
    P = slv.shape[0]
    w = lax.axis_index("s") * num_cores + lax.axis_index("c")

    pltpu.sync_copy(sl_ref.at[pl.ds(w * P, P)], slv)
    pltpu.sync_copy(sc_ref.at[pl.ds(w * P, P)], scv)

    nchunks = P // 16
    base_w = w * (L * C)
    per_row = P // _CHUNK

    def build(c, _):
        sl16 = slv[pl.ds(c * 16, 16)]
        sc16 = scv[pl.ds(c * 16, 16)]
        gb = (base_w + sl16 * C + sc16) >> 4
        col = (c % (_CHUNK // 16)) * 16
        rowoff = c // (_CHUNK // 16)
        for i in range(_PL):
            for z in range(2):
                k = 2 * i + z
                gidx[k * per_row + rowoff, pl.ds(col, 16)] = (
                    gb + (i * (C // _GR) + z))
        return 0

    lax.fori_loop(0, nchunks, build, 0)

    ndma = (2 * _PL * P) // _CHUNK
    group = 16
    handles = []
    for q in range(ndma):
        handles.append(pltpu.async_copy(
            xg_ref.at[gidx.at[q]],
            rows.at[pl.ds(q * _CHUNK, _CHUNK)], sem))
        if q >= group:
            handles[q - group].wait()
    for h in handles[ndma - group:]:
        h.wait()

    def extract(c, _):
        sc16 = scv[pl.ds(c * 16, 16)]
        o = sc16 & (_GR - 1)
        pv = c * 16 + lax.iota(jnp.int32, 16)
        for i in range(_PL):
            pv2i = pv + (2 * i) * P
            for j in range(_PC):
                u = o + j
                i0 = pv2i + (u >> 4) * P
                i1 = u & (_GR - 1)
                val = plsc.load_gather(rows, [i0, i1])
                patches[pl.ds((i * _PC + j) * P + c * 16, 16)] = val
        return 0

    lax.fori_loop(0, nchunks, extract, 0)

    pltpu.sync_copy(patches, out_ref.at[w])


def _mlp_tc_kernel(sl2_ref, pt_ref, w1p_ref, w1t_ref, b1_ref,
                   w2t_ref, b2_ref, out_ref):
    B, P = out_ref.shape[0], out_ref.shape[1]
    iv = lax.broadcasted_iota(jnp.int32, (P, _PL), 1).astype(jnp.float32)
    hi = lax.Precision.DEFAULT
    w1t = w1t_ref[...]
    w1p = w1p_ref[...]
    w2t = w2t_ref[...]
    b1v = b1_ref[...]
    b2v = b2_ref[...]
    for b in range(B):
        slv = sl2_ref[b].astype(jnp.float32)
        tv = (slv + iv) * (1.0 / _FS)
        acc = jnp.dot(tv, w1t, precision=hi) + b1v
        acc += lax.dot_general(pt_ref[b], w1p,
                               (((0,), (0,)), ((), ())), precision=hi)
        h = acc * jax.nn.sigmoid(acc)
        out_ref[b] = jnp.dot(h, w2t, precision=hi) + b2v


def kernel(x, start_indices_L, start_indices_C, W1, b1, W2, b2):
    B, L, C = x.shape
    P = start_indices_L.shape[1]
    D = W2.shape[0]
    BP = B * P

    sl = start_indices_L.astype(jnp.int32)
    sc = start_indices_C.astype(jnp.int32)
    xg = x.reshape(B * L * C // _GR, _GR)

    mesh = plsc.VectorSubcoreMesh(core_axis_name="c", subcore_axis_name="s",
                                  num_cores=2)
    sc_gather = functools.partial(
        pl.kernel, mesh=mesh,
        compiler_params=pltpu.CompilerParams(needs_layout_passes=False,
                                             use_tc_tiling_on_sc=False),
        out_type=jax.ShapeDtypeStruct((B, _PL * _PC * P), jnp.float32),
        scratch_types=[
            pltpu.VMEM((P,), jnp.int32),
            pltpu.VMEM((P,), jnp.int32),
            pltpu.VMEM((2 * _PL * P // _CHUNK, _CHUNK), jnp.int32),
            pltpu.VMEM((2 * _PL * P, _GR), jnp.float32),
            pltpu.VMEM((_PL * _PC * P,), jnp.float32),
            pltpu.SemaphoreType.DMA,
        ],
    )(functools.partial(_sc_gather_body, L, C))
    pt = sc_gather(xg, sl.reshape(BP), sc.reshape(BP))
    pt = pt.reshape(B, _PL * _PC, P)

    w1r = W1.reshape(D, _PL, 2 * _PC)
    w1p = w1r[:, :, :_PC].reshape(D, _PL * _PC).T
    w1t = w1r[:, :, _PC:].sum(axis=2).T
    w2t = W2.T
    b1r = b1.reshape(1, D)
    b2r = b2.reshape(1, D)

    out = pl.pallas_call(
        _mlp_tc_kernel,
        out_shape=jax.ShapeDtypeStruct((B, P, D), jnp.float32),
    )(sl.reshape(B, P, 1), pt, w1p, w1t, b1r, w2t, b2r)
    return out


# speedup vs baseline: 2.0032x; 1.0138x over previous
import functools

import jax
import jax.numpy as jnp
from jax import lax
from jax.experimental import pallas as pl
from jax.experimental.pallas import tpu as pltpu
from jax.experimental.pallas import tpu_sc as plsc

_PL, _PC = 8, 16
_FS = 100.0
_GR = 16
_CHUNK = 128
_NF = _PL * _PC + _PL


def _sc_gather_body(L, C, xg_ref, sl_ref, sc_ref, out_ref,
                    slv, scv, gidx, rows, patches, sem):
    num_cores = 2
    P = slv.shape[0]
    w = lax.axis_index("s") * num_cores + lax.axis_index("c")

    pltpu.sync_copy(sl_ref.at[pl.ds(w * P, P)], slv)
    pltpu.sync_copy(sc_ref.at[pl.ds(w * P, P)], scv)

    nchunks = P // 16
    base_w = w * (L * C)
    per_row = P // _CHUNK

    def build(c):
        sl16 = slv[pl.ds(c * 16, 16)]
        sc16 = scv[pl.ds(c * 16, 16)]
        gb = (base_w + sl16 * C + sc16) >> 4
        col = (c % (_CHUNK // 16)) * 16
        rowoff = c // (_CHUNK // 16)
        for i in range(_PL):
            for z in range(2):
                k = 2 * i + z
                gidx[k * per_row + rowoff, pl.ds(col, 16)] = (
                    gb + (i * (C // _GR) + z))
        tb = (slv[pl.ds(c * 16, 16)]).astype(jnp.float32)
        for i in range(_PL):
            patches[_PL * _PC + i, pl.ds(c * 16, 16)] = (
                (tb + float(i)) * (1.0 / _FS))

    def fire(half):
        return [pltpu.async_copy(
            xg_ref.at[gidx.at[2 * k + half]],
            rows.at[pl.ds((2 * k + half) * _CHUNK, _CHUNK)], sem)
            for k in range(_PL * 2)]

    def extract(c, _):
        sc16 = scv[pl.ds(c * 16, 16)]
        o = sc16 & (_GR - 1)
        pv = c * 16 + lax.iota(jnp.int32, 16)
        for i in range(_PL):
            pv2i = pv + (2 * i) * P
            for j in range(_PC):
                u = o + j
                i0 = pv2i + (u >> 4) * P
                i1 = u & (_GR - 1)
                val = plsc.load_gather(rows, [i0, i1])
                patches[i * _PC + j, pl.ds(c * 16, 16)] = val
        return 0

    half_c = nchunks // 2
    for c in range(half_c):
        build(c)
    h_even = fire(0)
    for c in range(half_c, nchunks):
        build(c)
    h_odd = fire(1)
    for h in h_even:
        h.wait()
    lax.fori_loop(0, half_c, extract, 0)
    for h in h_odd:
        h.wait()
    lax.fori_loop(half_c, nchunks, extract, 0)

    pltpu.sync_copy(patches, out_ref.at[:, pl.ds(w * P, P)])


def _mlp_tc_kernel(pt_ref, w1_ref, b1_ref, w2t_ref, b2_ref, out_ref):
    hi = lax.Precision.DEFAULT
    acc = lax.dot_general(pt_ref[...], w1_ref[...],
                          (((0,), (0,)), ((), ())), precision=hi)
    acc += b1_ref[...]
    h = acc * jax.nn.sigmoid(acc)
    out_ref[...] = jnp.dot(h, w2t_ref[...], precision=hi) + b2_ref[...]


def kernel(x, start_indices_L, start_indices_C, W1, b1, W2, b2):
    B, L, C = x.shape
    P = start_indices_L.shape[1]
    D = W2.shape[0]
    BP = B * P

    sl = start_indices_L.astype(jnp.int32)
    sc = start_indices_C.astype(jnp.int32)
    xg = x.reshape(B * L * C // _GR, _GR)

    mesh = plsc.VectorSubcoreMesh(core_axis_name="c", subcore_axis_name="s",
                                  num_cores=2)
    sc_gather = functools.partial(
        pl.kernel, mesh=mesh,
        compiler_params=pltpu.CompilerParams(needs_layout_passes=False,
                                             use_tc_tiling_on_sc=False),
        out_type=jax.ShapeDtypeStruct((_NF, BP), jnp.float32),
        scratch_types=[
            pltpu.VMEM((P,), jnp.int32),
            pltpu.VMEM((P,), jnp.int32),
            pltpu.VMEM((2 * _PL * P // _CHUNK, _CHUNK), jnp.int32),
            pltpu.VMEM((2 * _PL * P, _GR), jnp.float32),
            pltpu.VMEM((_NF, P), jnp.float32),
            pltpu.SemaphoreType.DMA,
        ],
    )(functools.partial(_sc_gather_body, L, C))
    pt = sc_gather(xg, sl.reshape(BP), sc.reshape(BP))

    w1r = W1.reshape(D, _PL, 2 * _PC)
    w1p = w1r[:, :, :_PC].reshape(D, _PL * _PC)
    w1t = w1r[:, :, _PC:].sum(axis=2)
    w1f = jnp.concatenate([w1p, w1t], axis=1).T
    w2t = W2.T
    b1r = b1.reshape(1, D)
    b2r = b2.reshape(1, D)

    out = pl.pallas_call(
        _mlp_tc_kernel,
        out_shape=jax.ShapeDtypeStruct((BP, D), jnp.float32),
    )(pt, w1f, b1r, w2t, b2r)
    return out.reshape(B, P, D)


# speedup vs baseline: 2.0301x; 1.0134x over previous
import functools

import jax
import jax.numpy as jnp
from jax import lax
from jax.experimental import pallas as pl
from jax.experimental.pallas import tpu as pltpu
from jax.experimental.pallas import tpu_sc as plsc

_PL, _PC = 8, 16
_FS = 100.0
_GR = 16
_CHUNK = 128
_NF = _PL * _PC + _PL


def _sc_gather_body(L, C, xg_ref, slsc_ref, out_ref,
                    slv, scv, gidx, rows, patches, sem):
    num_cores = 2
    P = slv.shape[0]
    BP = out_ref.shape[1]
    w = lax.axis_index("s") * num_cores + lax.axis_index("c")

    pltpu.sync_copy(slsc_ref.at[pl.ds(w * P, P)], slv)
    pltpu.sync_copy(slsc_ref.at[pl.ds(BP + w * P, P)], scv)

    nchunks = P // 16
    base_w = w * (L * C)
    per_row = P // _CHUNK

    def build(c, _):
        sl16 = slv[pl.ds(c * 16, 16)]
        sc16 = scv[pl.ds(c * 16, 16)]
        gb = (base_w + sl16 * C + sc16) >> 4
        col = (c % (_CHUNK // 16)) * 16
        rowoff = c // (_CHUNK // 16)
        for i in range(_PL):
            for z in range(2):
                k = 2 * i + z
                gidx[k * per_row + rowoff, pl.ds(col, 16)] = (
                    gb + (i * (C // _GR) + z))
        tb = sl16.astype(jnp.float32)
        for i in range(_PL):
            patches[_PL * _PC + i, pl.ds(c * 16, 16)] = (
                (tb + float(i)) * (1.0 / _FS))
        return 0

    def fire(half):
        return [pltpu.async_copy(
            xg_ref.at[gidx.at[2 * k + half]],
            rows.at[pl.ds((2 * k + half) * _CHUNK, _CHUNK)], sem)
            for k in range(_PL * 2)]

    def extract(c, _):
        sc16 = scv[pl.ds(c * 16, 16)]
        o = sc16 & (_GR - 1)
        pv = c * 16 + lax.iota(jnp.int32, 16)
        for i in range(_PL):
            pv2i = pv + (2 * i) * P
            for j in range(_PC):
                u = o + j
                i0 = pv2i + (u >> 4) * P
                i1 = u & (_GR - 1)
                val = plsc.load_gather(rows, [i0, i1])
                patches[i * _PC + j, pl.ds(c * 16, 16)] = val
        return 0

    half_c = nchunks // 2
    lax.fori_loop(0, half_c, build, 0)
    h_even = fire(0)
    lax.fori_loop(half_c, nchunks, build, 0)
    h_odd = fire(1)
    for h in h_even:
        h.wait()
    lax.fori_loop(0, half_c, extract, 0)
    for h in h_odd:
        h.wait()
    lax.fori_loop(half_c, nchunks, extract, 0)

    pltpu.sync_copy(patches, out_ref.at[:, pl.ds(w * P, P)])


def _mlp_tc_kernel(pt_ref, w1_ref, b1_ref, w2t_ref, b2_ref, out_ref):
    B, P, D = out_ref.shape
    hi = lax.Precision.DEFAULT
    acc = lax.dot_general(pt_ref[...], w1_ref[...],
                          (((0,), (0,)), ((), ())), precision=hi)
    acc += b1_ref[...]
    h = acc * jax.nn.sigmoid(acc)
    res = jnp.dot(h, w2t_ref[...], precision=hi) + b2_ref[...]
    out_ref[...] = res.reshape(B, P, D)


def kernel(x, start_indices_L, start_indices_C, W1, b1, W2, b2):
    B, L, C = x.shape
    P = start_indices_L.shape[1]
    D = W2.shape[0]
    BP = B * P

    sl = start_indices_L.astype(jnp.int32)
    sc = start_indices_C.astype(jnp.int32)
    xg = x.reshape(B * L * C // _GR, _GR)

    mesh = plsc.VectorSubcoreMesh(core_axis_name="c", subcore_axis_name="s",
                                  num_cores=2)
    sc_gather = functools.partial(
        pl.kernel, mesh=mesh,
        compiler_params=pltpu.CompilerParams(needs_layout_passes=False,
                                             use_tc_tiling_on_sc=False),
        out_type=jax.ShapeDtypeStruct((_NF, BP), jnp.float32),
        scratch_types=[
            pltpu.VMEM((P,), jnp.int32),
            pltpu.VMEM((P,), jnp.int32),
            pltpu.VMEM((2 * _PL * P // _CHUNK, _CHUNK), jnp.int32),
            pltpu.VMEM((2 * _PL * P, _GR), jnp.float32),
            pltpu.VMEM((_NF, P), jnp.float32),
            pltpu.SemaphoreType.DMA,
        ],
    )(functools.partial(_sc_gather_body, L, C))
    slsc = jnp.concatenate([sl.reshape(BP), sc.reshape(BP)])
    pt = sc_gather(xg, slsc)

    w1r = W1.reshape(D, _PL, 2 * _PC)
    w1p = w1r[:, :, :_PC].reshape(D, _PL * _PC)
    w1t = w1r[:, :, _PC:].sum(axis=2)
    w1f = jnp.concatenate([w1p, w1t], axis=1).T
    w2t = W2.T
    b1r = b1.reshape(1, D)
    b2r = b2.reshape(1, D)

    out = pl.pallas_call(
        _mlp_tc_kernel,
        out_shape=jax.ShapeDtypeStruct((B, P, D), jnp.float32),
    )(pt, w1f, b1r, w2t, b2r)
    return out
